# baseline (speedup 1.0000x reference)
import functools

import jax
import jax.numpy as jnp
from jax import lax
from jax.experimental import pallas as pl
from jax.experimental.pallas import tpu as pltpu
from jax.experimental.pallas import tpu_sc as plsc

_B, _Q, _D = 2, 2048, 256
_G, _P = 4, 8
_GP = _G * _P
_N, _C, _HF, _WF = 6, 256, 64, 176
_Cg = _C // _G
_IMG_H, _IMG_W = 256, 704
_EPS = 1e-5
_ROWS2 = _B * _N * _HF * _WF * 2

_QB = 512
_NP = _B * _Q * _GP
_NW = 32
_CH = 64


def _transpose_body(x_ref, o_ref):
    o_ref[0] = x_ref[0].T


def _build_table(mlvl_feats):
    feats = mlvl_feats.reshape(_B * _N, _C, _HF * _WF)
    hwb = 1024
    out = pl.pallas_call(
        _transpose_body,
        grid=(_B * _N, (_HF * _WF) // hwb),
        in_specs=[pl.BlockSpec((1, _C, hwb), lambda i, j: (i, 0, j))],
        out_specs=pl.BlockSpec((1, hwb, _C), lambda i, j: (i, j, 0)),
        out_shape=jax.ShapeDtypeStruct((_B * _N, _HF * _WF, _C), jnp.float32),
    )(feats)
    return out.reshape(_ROWS2, 2 * _Cg)


def _mm_body(q_ref, w_ref, o_ref):
    o_ref[0] = jnp.dot(q_ref[0].astype(jnp.bfloat16),
                       w_ref[...].astype(jnp.bfloat16),
                       preferred_element_type=jnp.float32)


def _offsets(query, W_off):
    w_pad = jnp.concatenate(
        [W_off, jnp.zeros((128 - 3 * _GP, _D), jnp.float32)], axis=0)
    return pl.pallas_call(
        _mm_body,
        grid=(_B, _Q // _QB),
        in_specs=[
            pl.BlockSpec((1, _QB, _D), lambda b, q: (b, q, 0)),
            pl.BlockSpec((_D, 128), lambda b, q: (0, 0)),
        ],
        out_specs=pl.BlockSpec((1, _QB, 128), lambda b, q: (b, q, 0)),
        out_shape=jax.ShapeDtypeStruct((_B, _Q, 128), jnp.float32),
    )(query, w_pad.T)[:, :, :3 * _GP]


def _index_body(sx_ref, sy_ref, ms_ref, sn_ref,
                i00_ref, i01_ref, i10_ref, i11_ref,
                w00_ref, w01_ref, w10_ref, w11_ref):
    b = pl.program_id(0)
    sx = sx_ref[0]
    sy = sy_ref[0]
    m_sel = ms_ref[0]
    sn = sn_ref[0]
    fx = sx * float(_WF) - 0.5
    fy = sy * float(_HF) - 0.5
    x0f = jnp.floor(fx)
    y0f = jnp.floor(fy)
    wx1 = fx - x0f
    wy1 = fy - y0f
    inbx0 = (x0f >= 0.0) & (x0f <= float(_WF - 1))
    inbx1 = (x0f >= -1.0) & (x0f <= float(_WF - 2))
    inby0 = (y0f >= 0.0) & (y0f <= float(_HF - 1))
    inby1 = (y0f >= -1.0) & (y0f <= float(_HF - 2))
    x0i = jnp.clip(x0f, 0.0, float(_WF - 1)).astype(jnp.int32)
    x1i = jnp.clip(x0f + 1.0, 0.0, float(_WF - 1)).astype(jnp.int32)
    y0i = jnp.clip(y0f, 0.0, float(_HF - 1)).astype(jnp.int32)
    y1i = jnp.clip(y0f + 1.0, 0.0, float(_HF - 1)).astype(jnp.int32)

    g_half = lax.broadcasted_iota(jnp.int32, (_QB, _GP), 1) // (2 * _P)
    bn = (b * _N + sn) * (_HF * _WF * 2)
    i00_ref[0] = bn + y0i * (_WF * 2) + x0i * 2 + g_half
    i01_ref[0] = bn + y0i * (_WF * 2) + x1i * 2 + g_half
    i10_ref[0] = bn + y1i * (_WF * 2) + x0i * 2 + g_half
    i11_ref[0] = bn + y1i * (_WF * 2) + x1i * 2 + g_half

    wxa = 1.0 - wx1
    wya = 1.0 - wy1
    w00_ref[0] = wya * wxa * inbx0.astype(jnp.float32) * inby0.astype(jnp.float32) * m_sel
    w01_ref[0] = wya * wx1 * inbx1.astype(jnp.float32) * inby0.astype(jnp.float32) * m_sel
    w10_ref[0] = wy1 * wxa * inbx0.astype(jnp.float32) * inby1.astype(jnp.float32) * m_sel
    w11_ref[0] = wy1 * wx1 * inbx1.astype(jnp.float32) * inby1.astype(jnp.float32) * m_sel


def _select_views(query, reference_points, lidar2img, W_off, b_off):
    off = _offsets(query, W_off) + b_off
    off = off.reshape(_B, _Q, _G, _P, 3)
    ref = reference_points[:, :, None, None, :]
    lo = jnp.array([-51.2, -51.2, -5.0], dtype=jnp.float32)
    hi = jnp.array([51.2, 51.2, 3.0], dtype=jnp.float32)
    ref = ref * (hi - lo) + lo
    pts = ref + off
    pts = pts.reshape(_B, _Q, _GP, 3)
    pts_h = jnp.concatenate([pts, jnp.ones_like(pts[..., :1])], axis=-1)
    cam = jnp.einsum('bnij,bqpj->bnqpi', lidar2img, pts_h)
    homo = cam[..., 2:3]
    homo_nz = jnp.maximum(homo, _EPS)
    xy = cam[..., 0:2] / homo_nz
    x01 = xy[..., 0] / _IMG_W
    y01 = xy[..., 1] / _IMG_H
    valid = ((homo[..., 0] > _EPS) & (x01 > 0.0) & (x01 < 1.0)
             & (y01 > 0.0) & (y01 < 1.0))
    valid_f = valid.astype(jnp.float32)
    i_view = jnp.argmax(valid_f, axis=1)
    x01t = jnp.transpose(x01, (0, 2, 3, 1))
    y01t = jnp.transpose(y01, (0, 2, 3, 1))
    vft = jnp.transpose(valid_f, (0, 2, 3, 1))
    iv = i_view[..., None]
    x_sel = jnp.take_along_axis(x01t, iv, axis=-1)[..., 0]
    y_sel = jnp.take_along_axis(y01t, iv, axis=-1)[..., 0]
    m_sel = jnp.take_along_axis(vft, iv, axis=-1)[..., 0]
    return x_sel, y_sel, m_sel, i_view.astype(jnp.int32)


def _build_indices(x_sel, y_sel, m_sel, i_view):
    grid = (_B, _Q // _QB)
    in_spec = pl.BlockSpec((1, _QB, _GP), lambda b, q: (b, q, 0))
    out_spec = pl.BlockSpec((1, _QB, _GP), lambda b, q: (b, q, 0))
    ishape = jax.ShapeDtypeStruct((_B, _Q, _GP), jnp.int32)
    wshape = jax.ShapeDtypeStruct((_B, _Q, _GP), jnp.float32)
    outs = pl.pallas_call(
        _index_body,
        grid=grid,
        in_specs=[in_spec] * 4,
        out_specs=[out_spec] * 8,
        out_shape=[ishape] * 4 + [wshape] * 4,
    )(x_sel, y_sel, m_sel, i_view)
    return tuple(o.reshape(_NP) for o in outs)


def _sc_gather4(table, i00, i01, i10, i11):
    mesh = plsc.VectorSubcoreMesh(core_axis_name="c", subcore_axis_name="s")
    rshape = jax.ShapeDtypeStruct((_NP, 2 * _Cg), jnp.float32)

    wpts = _NP // _NW
    rowbuf = pltpu.VMEM((_CH, 2 * _Cg), jnp.float32)

    @functools.partial(
        pl.kernel,
        mesh=mesh,
        out_type=(rshape, rshape, rshape, rshape),
        scratch_types=[
            pltpu.VMEM((wpts,), jnp.int32),
            pltpu.VMEM((wpts,), jnp.int32),
            pltpu.VMEM((wpts,), jnp.int32),
            pltpu.VMEM((wpts,), jnp.int32),
            rowbuf, rowbuf, rowbuf, rowbuf,
            rowbuf, rowbuf, rowbuf, rowbuf,
            pltpu.SemaphoreType.DMA,
            pltpu.SemaphoreType.DMA,
        ],
    )
    def sample(table_hbm, i00_hbm, i01_hbm, i10_hbm, i11_hbm,
               o0_hbm, o1_hbm, o2_hbm, o3_hbm,
               ia0, ia1, ia2, ia3,
               a0, a1, a2, a3, b0, b1, b2, b3, semA, semB):
        wid = lax.axis_index("s") * 2 + lax.axis_index("c")
        wbase = wid * wpts
        pltpu.sync_copy(i00_hbm.at[pl.ds(wbase, wpts)], ia0)
        pltpu.sync_copy(i01_hbm.at[pl.ds(wbase, wpts)], ia1)
        pltpu.sync_copy(i10_hbm.at[pl.ds(wbase, wpts)], ia2)
        pltpu.sync_copy(i11_hbm.at[pl.ds(wbase, wpts)], ia3)

        def pair(i, carry):
            lo = (2 * i) * _CH
            hi = (2 * i + 1) * _CH
            cA = [pltpu.async_copy(table_hbm.at[ia.at[pl.ds(lo, _CH)]], r, semA)
                  for ia, r in ((ia0, a0), (ia1, a1), (ia2, a2), (ia3, a3))]
            cB = [pltpu.async_copy(table_hbm.at[ia.at[pl.ds(hi, _CH)]], r, semB)
                  for ia, r in ((ia0, b0), (ia1, b1), (ia2, b2), (ia3, b3))]
            for c in cA:
                c.wait()
            for r, o in ((a0, o0_hbm), (a1, o1_hbm), (a2, o2_hbm), (a3, o3_hbm)):
                pltpu.sync_copy(r, o.at[pl.ds(wbase + lo, _CH)])
            for c in cB:
                c.wait()
            for r, o in ((b0, o0_hbm), (b1, o1_hbm), (b2, o2_hbm), (b3, o3_hbm)):
                pltpu.sync_copy(r, o.at[pl.ds(wbase + hi, _CH)])
            return carry

        lax.fori_loop(0, wpts // (2 * _CH), pair, 0)

    return sample(table, i00, i01, i10, i11)


_PB = 512


def _combine_body(r0_ref, r1_ref, r2_ref, r3_ref,
                  w0_ref, w1_ref, w2_ref, w3_ref, o_ref):
    row = lax.broadcasted_iota(jnp.int32, (_PB, _Cg), 0)
    lo_half = ((row // _P) % 2) == 0

    def half(r_ref):
        return jnp.where(lo_half, r_ref[:, :_Cg], r_ref[:, _Cg:])

    o_ref[...] = (w0_ref[...] * half(r0_ref) + w1_ref[...] * half(r1_ref)
                  + w2_ref[...] * half(r2_ref) + w3_ref[...] * half(r3_ref))


def _combine(r00, r01, r10, r11, w00, w01, w10, w11):
    rspec = pl.BlockSpec((_PB, 2 * _Cg), lambda i: (i, 0))
    wspec = pl.BlockSpec((_PB, 1), lambda i: (i, 0))
    ws = [w.reshape(_NP, 1) for w in (w00, w01, w10, w11)]
    return pl.pallas_call(
        _combine_body,
        grid=(_NP // _PB,),
        in_specs=[rspec] * 4 + [wspec] * 4,
        out_specs=pl.BlockSpec((_PB, _Cg), lambda i: (i, 0)),
        out_shape=jax.ShapeDtypeStruct((_NP, _Cg), jnp.float32),
    )(r00, r01, r10, r11, *ws)


def kernel(query, mlvl_feats, reference_points, lidar2img, W_off, b_off):
    table = _build_table(mlvl_feats)
    x_sel, y_sel, m_sel, i_view = _select_views(
        query, reference_points, lidar2img, W_off, b_off)
    i00, i01, i10, i11, w00, w01, w10, w11 = _build_indices(
        x_sel, y_sel, m_sel, i_view)
    r00, r01, r10, r11 = _sc_gather4(table, i00, i01, i10, i11)
    out = _combine(r00, r01, r10, r11, w00, w01, w10, w11)
    return out.reshape(_B, _Q, _G, _P, _Cg)


# speedup vs baseline: 1.2035x; 1.2035x over previous
import functools

import jax
import jax.numpy as jnp
from jax import lax
from jax.experimental import pallas as pl
from jax.experimental.pallas import tpu as pltpu
from jax.experimental.pallas import tpu_sc as plsc

_B, _Q, _D = 2, 2048, 256
_G, _P = 4, 8
_GP = _G * _P
_N, _C, _HF, _WF = 6, 256, 64, 176
_Cg = _C // _G
_IMG_H, _IMG_W = 256, 704
_EPS = 1e-5
_ROWS2 = _B * _N * _HF * _WF * 2

_QB = 512
_NP = _B * _Q * _GP
_NW = 32
_CH = 64


def _transpose_body(x_ref, o_ref):
    o_ref[0] = x_ref[0].T


def _build_table(mlvl_feats):
    feats = mlvl_feats.reshape(_B * _N, _C, _HF * _WF)
    hwb = 1024
    out = pl.pallas_call(
        _transpose_body,
        grid=(_B * _N, (_HF * _WF) // hwb),
        in_specs=[pl.BlockSpec((1, _C, hwb), lambda i, j: (i, 0, j))],
        out_specs=pl.BlockSpec((1, hwb, _C), lambda i, j: (i, j, 0)),
        out_shape=jax.ShapeDtypeStruct((_B * _N, _HF * _WF, _C), jnp.float32),
    )(feats)
    return out.reshape(_ROWS2, 2 * _Cg)


def _mm_body(q_ref, w_ref, o_ref):
    o_ref[0] = jnp.dot(q_ref[0].astype(jnp.bfloat16),
                       w_ref[...].astype(jnp.bfloat16),
                       preferred_element_type=jnp.float32)


def _offsets(query, W_off):
    w_pad = jnp.concatenate(
        [W_off, jnp.zeros((128 - 3 * _GP, _D), jnp.float32)], axis=0)
    return pl.pallas_call(
        _mm_body,
        grid=(_B, _Q // _QB),
        in_specs=[
            pl.BlockSpec((1, _QB, _D), lambda b, q: (b, q, 0)),
            pl.BlockSpec((_D, 128), lambda b, q: (0, 0)),
        ],
        out_specs=pl.BlockSpec((1, _QB, 128), lambda b, q: (b, q, 0)),
        out_shape=jax.ShapeDtypeStruct((_B, _Q, 128), jnp.float32),
    )(query, w_pad.T)[:, :, :3 * _GP]


def _index_body(sx_ref, sy_ref, ms_ref, sn_ref,
                i00_ref, i01_ref, i10_ref, i11_ref,
                w00_ref, w01_ref, w10_ref, w11_ref):
    b = pl.program_id(0)
    sx = sx_ref[0]
    sy = sy_ref[0]
    m_sel = ms_ref[0]
    sn = sn_ref[0]
    fx = sx * float(_WF) - 0.5
    fy = sy * float(_HF) - 0.5
    x0f = jnp.floor(fx)
    y0f = jnp.floor(fy)
    wx1 = fx - x0f
    wy1 = fy - y0f
    inbx0 = (x0f >= 0.0) & (x0f <= float(_WF - 1))
    inbx1 = (x0f >= -1.0) & (x0f <= float(_WF - 2))
    inby0 = (y0f >= 0.0) & (y0f <= float(_HF - 1))
    inby1 = (y0f >= -1.0) & (y0f <= float(_HF - 2))
    x0i = jnp.clip(x0f, 0.0, float(_WF - 1)).astype(jnp.int32)
    x1i = jnp.clip(x0f + 1.0, 0.0, float(_WF - 1)).astype(jnp.int32)
    y0i = jnp.clip(y0f, 0.0, float(_HF - 1)).astype(jnp.int32)
    y1i = jnp.clip(y0f + 1.0, 0.0, float(_HF - 1)).astype(jnp.int32)

    g_half = lax.broadcasted_iota(jnp.int32, (_QB, _GP), 1) // (2 * _P)
    bn = (b * _N + sn) * (_HF * _WF * 2)
    i00_ref[0] = bn + y0i * (_WF * 2) + x0i * 2 + g_half
    i01_ref[0] = bn + y0i * (_WF * 2) + x1i * 2 + g_half
    i10_ref[0] = bn + y1i * (_WF * 2) + x0i * 2 + g_half
    i11_ref[0] = bn + y1i * (_WF * 2) + x1i * 2 + g_half

    wxa = 1.0 - wx1
    wya = 1.0 - wy1
    w00_ref[0] = wya * wxa * inbx0.astype(jnp.float32) * inby0.astype(jnp.float32) * m_sel
    w01_ref[0] = wya * wx1 * inbx1.astype(jnp.float32) * inby0.astype(jnp.float32) * m_sel
    w10_ref[0] = wy1 * wxa * inbx0.astype(jnp.float32) * inby1.astype(jnp.float32) * m_sel
    w11_ref[0] = wy1 * wx1 * inbx1.astype(jnp.float32) * inby1.astype(jnp.float32) * m_sel


def _select_views(query, reference_points, lidar2img, W_off, b_off):
    off = _offsets(query, W_off) + b_off
    off = off.reshape(_B, _Q, _G, _P, 3)
    ref = reference_points[:, :, None, None, :]
    lo = jnp.array([-51.2, -51.2, -5.0], dtype=jnp.float32)
    hi = jnp.array([51.2, 51.2, 3.0], dtype=jnp.float32)
    ref = ref * (hi - lo) + lo
    pts = ref + off
    pts = pts.reshape(_B, _Q, _GP, 3)
    pts_h = jnp.concatenate([pts, jnp.ones_like(pts[..., :1])], axis=-1)
    cam = jnp.einsum('bnij,bqpj->bnqpi', lidar2img, pts_h)
    homo = cam[..., 2:3]
    homo_nz = jnp.maximum(homo, _EPS)
    xy = cam[..., 0:2] / homo_nz
    x01 = xy[..., 0] / _IMG_W
    y01 = xy[..., 1] / _IMG_H
    valid = ((homo[..., 0] > _EPS) & (x01 > 0.0) & (x01 < 1.0)
             & (y01 > 0.0) & (y01 < 1.0))
    valid_f = valid.astype(jnp.float32)
    i_view = jnp.argmax(valid_f, axis=1)
    x01t = jnp.transpose(x01, (0, 2, 3, 1))
    y01t = jnp.transpose(y01, (0, 2, 3, 1))
    vft = jnp.transpose(valid_f, (0, 2, 3, 1))
    iv = i_view[..., None]
    x_sel = jnp.take_along_axis(x01t, iv, axis=-1)[..., 0]
    y_sel = jnp.take_along_axis(y01t, iv, axis=-1)[..., 0]
    m_sel = jnp.take_along_axis(vft, iv, axis=-1)[..., 0]
    return x_sel, y_sel, m_sel, i_view.astype(jnp.int32)


def _build_indices(x_sel, y_sel, m_sel, i_view):
    grid = (_B, _Q // _QB)
    in_spec = pl.BlockSpec((1, _QB, _GP), lambda b, q: (b, q, 0))
    out_spec = pl.BlockSpec((1, _QB, _GP), lambda b, q: (b, q, 0))
    ishape = jax.ShapeDtypeStruct((_B, _Q, _GP), jnp.int32)
    wshape = jax.ShapeDtypeStruct((_B, _Q, _GP), jnp.float32)
    outs = pl.pallas_call(
        _index_body,
        grid=grid,
        in_specs=[in_spec] * 4,
        out_specs=[out_spec] * 8,
        out_shape=[ishape] * 4 + [wshape] * 4,
    )(x_sel, y_sel, m_sel, i_view)
    return tuple(o.reshape(_NP) for o in outs)


_NCH = _NP // (_NW * _CH)


def _sc_gather_combine(table, i00, i01, i10, i11, w00, w01, w10, w11):
    mesh = plsc.VectorSubcoreMesh(core_axis_name="c", subcore_axis_name="s")
    rowbuf = pltpu.VMEM((_CH, 2 * _Cg), jnp.float32)

    @functools.partial(
        pl.kernel,
        mesh=mesh,
        out_type=jax.ShapeDtypeStruct((_NP, _Cg), jnp.float32),
        scratch_types=[
            pltpu.VMEM((_CH,), jnp.int32),
            pltpu.VMEM((_CH,), jnp.int32),
            pltpu.VMEM((_CH,), jnp.int32),
            pltpu.VMEM((_CH,), jnp.int32),
            pltpu.VMEM((_CH,), jnp.float32),
            pltpu.VMEM((_CH,), jnp.float32),
            pltpu.VMEM((_CH,), jnp.float32),
            pltpu.VMEM((_CH,), jnp.float32),
            rowbuf, rowbuf, rowbuf, rowbuf,
            pltpu.VMEM((_CH, _Cg), jnp.float32),
            pltpu.SemaphoreType.DMA,
        ],
    )
    def sample(table_hbm, i00_hbm, i01_hbm, i10_hbm, i11_hbm,
               w00_hbm, w01_hbm, w10_hbm, w11_hbm, out_hbm,
               iv0, iv1, iv2, iv3, wv0, wv1, wv2, wv3,
               r0, r1, r2, r3, ob, sem):
        wid = lax.axis_index("s") * 2 + lax.axis_index("c")

        def chunk(ch, carry):
            base = wid * (_NCH * _CH) + ch * _CH
            pltpu.sync_copy(i00_hbm.at[pl.ds(base, _CH)], iv0)
            pltpu.sync_copy(i01_hbm.at[pl.ds(base, _CH)], iv1)
            pltpu.sync_copy(i10_hbm.at[pl.ds(base, _CH)], iv2)
            pltpu.sync_copy(i11_hbm.at[pl.ds(base, _CH)], iv3)
            pltpu.sync_copy(w00_hbm.at[pl.ds(base, _CH)], wv0)
            pltpu.sync_copy(w01_hbm.at[pl.ds(base, _CH)], wv1)
            pltpu.sync_copy(w10_hbm.at[pl.ds(base, _CH)], wv2)
            pltpu.sync_copy(w11_hbm.at[pl.ds(base, _CH)], wv3)
            c0 = pltpu.async_copy(table_hbm.at[iv0], r0, sem)
            c1 = pltpu.async_copy(table_hbm.at[iv1], r1, sem)
            c2 = pltpu.async_copy(table_hbm.at[iv2], r2, sem)
            c3 = pltpu.async_copy(table_hbm.at[iv3], r3, sem)
            c0.wait()
            c1.wait()
            c2.wait()
            c3.wait()

            def group(g, carry2):
                w0g = wv0[pl.ds(g * 16, 16)]
                w1g = wv1[pl.ds(g * 16, 16)]
                w2g = wv2[pl.ds(g * 16, 16)]
                w3g = wv3[pl.ds(g * 16, 16)]
                dn = lax.GatherDimensionNumbers(
                    offset_dims=(), collapsed_slice_dims=(0,),
                    start_index_map=(0,))
                for j in range(16):
                    jv = jnp.full((16, 1), j, jnp.int32)
                    splat = lambda w: lax.gather(
                        w, jv, dn, (1,),
                        mode=lax.GatherScatterMode.PROMISE_IN_BOUNDS)
                    a0 = splat(w0g)
                    a1 = splat(w1g)
                    a2 = splat(w2g)
                    a3 = splat(w3g)
                    p = g * 16 + j
                    cb = ((j // _P) % 2) * _Cg
                    for jj in range(_Cg // 16):
                        s = pl.ds(cb + jj * 16, 16)
                        so = pl.ds(jj * 16, 16)
                        acc = (a0 * r0[p, s] + a1 * r1[p, s]
                               + a2 * r2[p, s] + a3 * r3[p, s])
                        ob[p, so] = acc
                return carry2

            lax.fori_loop(0, _CH // 16, group, 0)
            pltpu.sync_copy(ob, out_hbm.at[pl.ds(base, _CH)])
            return carry

        lax.fori_loop(0, _NCH, chunk, 0)

    return sample(table, i00, i01, i10, i11, w00, w01, w10, w11)


def kernel(query, mlvl_feats, reference_points, lidar2img, W_off, b_off):
    table = _build_table(mlvl_feats)
    x_sel, y_sel, m_sel, i_view = _select_views(
        query, reference_points, lidar2img, W_off, b_off)
    i00, i01, i10, i11, w00, w01, w10, w11 = _build_indices(
        x_sel, y_sel, m_sel, i_view)
    out = _sc_gather_combine(table, i00, i01, i10, i11, w00, w01, w10, w11)
    return out.reshape(_B, _Q, _G, _P, _Cg)
